# baseline (speedup 1.0000x reference)
import functools

import jax
import jax.numpy as jnp
from jax import lax
from jax.experimental import pallas as pl
from jax.experimental.pallas import tpu as pltpu
from jax.experimental.pallas import tpu_sc as plsc

N = 10000
E = 320000
D = 128
H = 64
NG = 16
NCLS = 10

NCORES = 2
NSUB = 16
NW = NCORES * NSUB
EPW = E // NW
K = 80
NCH = EPW // K
HT = 80
RPS = N // NSUB


def _dense1_body(x_ref, w_ref, wp_ref, as_ref, ad_ref,
                 hb_ref, aso_ref, ado_ref):
    h = jnp.dot(x_ref[...], w_ref[...], preferred_element_type=jnp.float32)
    hp = jnp.dot(x_ref[...], wp_ref[...], preferred_element_type=jnp.float32)
    hb_ref[...] = hp.astype(jnp.bfloat16)
    aso_ref[...] = jnp.dot(h, as_ref[...], preferred_element_type=jnp.float32)
    ado_ref[...] = jnp.dot(h, ad_ref[...], preferred_element_type=jnp.float32)


_dense1 = pl.pallas_call(
    _dense1_body,
    out_shape=[
        jax.ShapeDtypeStruct((N, H), jnp.bfloat16),
        jax.ShapeDtypeStruct((N, 1), jnp.float32),
        jax.ShapeDtypeStruct((N, 1), jnp.float32),
    ],
)


def _dense2_body(acc_ref, b_ref, w_ref, wp_ref, as_ref, ad_ref,
                 hb_ref, aso_ref, ado_ref):
    u = acc_ref[0] + acc_ref[1]
    o = u[:, :H] / (u[:, H:H + 1] + 1e-16) + b_ref[...]
    g = jnp.maximum(o, 0.0)
    h = jnp.dot(g, w_ref[...], preferred_element_type=jnp.float32)
    hp = jnp.dot(g, wp_ref[...], preferred_element_type=jnp.float32)
    hb_ref[...] = hp.astype(jnp.bfloat16)
    aso_ref[...] = jnp.dot(h, as_ref[...], preferred_element_type=jnp.float32)
    ado_ref[...] = jnp.dot(h, ad_ref[...], preferred_element_type=jnp.float32)


_dense2 = pl.pallas_call(
    _dense2_body,
    out_shape=[
        jax.ShapeDtypeStruct((N, H), jnp.bfloat16),
        jax.ShapeDtypeStruct((N, 1), jnp.float32),
        jax.ShapeDtypeStruct((N, 1), jnp.float32),
    ],
)


def _final_body(acc_ref, b_ref, batch_ref, wl_ref, bl_ref, wc_ref, bc_ref,
                out_ref):
    u = acc_ref[0] + acc_ref[1]
    o = u[:, :H] / (u[:, H:H + 1] + 1e-16) + b_ref[...]
    gid = lax.broadcasted_iota(jnp.int32, (NG, N), 0)
    m = (gid == batch_ref[...]).astype(jnp.float32)
    sums = jnp.dot(m, o, preferred_element_type=jnp.float32)
    cnt = jnp.sum(m, axis=1, keepdims=True)
    pooled = sums / jnp.maximum(cnt, 1.0)
    z = jnp.maximum(
        jnp.dot(pooled, wl_ref[...], preferred_element_type=jnp.float32)
        + bl_ref[...], 0.0)
    logits = jnp.dot(z, wc_ref[...],
                     preferred_element_type=jnp.float32) + bc_ref[...]
    mx = jnp.max(logits, axis=1, keepdims=True)
    lse = mx + jnp.log(jnp.sum(jnp.exp(logits - mx), axis=1, keepdims=True))
    out_ref[...] = logits - lse


_final = pl.pallas_call(
    _final_body,
    out_shape=jax.ShapeDtypeStruct((NG, NCLS), jnp.float32),
)


def _lane_bcast(v, lane):
    idx = jnp.full((16, 1), lane, jnp.int32)
    return lax.gather(
        v, idx,
        lax.GatherDimensionNumbers(
            offset_dims=(), collapsed_slice_dims=(0,), start_index_map=(0,)),
        (1,), mode=lax.GatherScatterMode.PROMISE_IN_BOUNDS)


def _edge_body(hb_hbm, asrc_hbm, adst_hbm, src_hbm, dst_hbm, zeros_hbm,
               out_hbm, asrc_v, adst_v, srcs_v, dsts_v,
               raw0, raw1, raw2, rows0, rows1, rows2, accum,
               sg0, sg1, sg2, ss0, ss1, ss2):
    c = lax.axis_index("c")
    s = lax.axis_index("s")
    wid = s * NCORES + c
    d0 = pltpu.async_copy(zeros_hbm, accum.at[pl.ds(s * RPS, RPS)], sg0)
    d1 = pltpu.async_copy(asrc_hbm, asrc_v, sg0)
    d2 = pltpu.async_copy(adst_hbm, adst_v, sg0)
    d3 = pltpu.async_copy(src_hbm.at[wid], srcs_v, sg0)
    d4 = pltpu.async_copy(dst_hbm.at[wid], dsts_v, sg0)
    d0.wait(); d1.wait(); d2.wait(); d3.wait(); d4.wait()
    plsc.subcore_barrier()

    raw = (raw0, raw1, raw2)
    rows = (rows0, rows1, rows2)
    sg = (sg0, sg1, sg2)
    ss = (ss0, ss1, ss2)

    def gather(j, b):
        return pltpu.async_copy(hb_hbm.at[srcs_v.at[j]], raw[b], sg[b])

    def wait_gather(j, b):
        pltpu.make_async_copy(hb_hbm.at[srcs_v.at[j]], raw[b], sg[b]).wait()

    def wait_scatter(j, b):
        pltpu.make_async_copy(rows[b], accum.at[dsts_v.at[j]], ss[b]).wait()

    def wvals(j):
        ws = []
        for i in range(K // 16):
            sv = srcs_v[j, pl.ds(i * 16, 16)]
            dv = dsts_v[j, pl.ds(i * 16, 16)]
            e = plsc.load_gather(asrc_v, [sv]) + plsc.load_gather(adst_v, [dv])
            e = jnp.maximum(e, e * 0.2)
            ws.append(jnp.exp(e))
        return ws

    def phase(j, b, swait):
        gather(j + 2, (b + 2) % 3)
        ws = wvals(j)
        wait_gather(j, b)
        if swait:
            wait_scatter(j - 3, b)
        mhi = jnp.full((16,), -65536, jnp.int32)
        for r in range(K):
            wsp = _lane_bcast(ws[r // 16], r % 16)
            for q in range(2):
                v = plsc.bitcast(raw[b][r, pl.ds(q * 32, 32)], jnp.int32)
                lo = plsc.bitcast(lax.shift_left(v, 16), jnp.float32)
                hi = plsc.bitcast(lax.bitwise_and(v, mhi), jnp.float32)
                rows[b][r, pl.ds(q * 32, 16)] = lo * wsp
                rows[b][r, pl.ds(q * 32 + 16, 16)] = hi * wsp
            rows[b][r, pl.ds(H, 16)] = wsp
        pltpu.async_copy(rows[b], accum.at[dsts_v.at[j]], ss[b], add=True)

    gather(0, 0)
    gather(1, 1)
    phase(0, 0, False)
    phase(1, 1, False)
    phase(2, 2, False)
    phase(3, 0, True)
    phase(4, 1, True)

    def triple(p, carry):
        j = 3 * p + 2
        phase(j, 2, True)
        phase(j + 1, 0, True)
        phase(j + 2, 1, True)
        return carry

    lax.fori_loop(1, (NCH - 2) // 3, triple, 0)
    wait_gather(NCH, NCH % 3)
    wait_gather(NCH + 1, (NCH + 1) % 3)
    wait_scatter(NCH - 3, (NCH - 3) % 3)
    wait_scatter(NCH - 2, (NCH - 2) % 3)
    wait_scatter(NCH - 1, (NCH - 1) % 3)

    plsc.subcore_barrier()
    pltpu.sync_copy(accum.at[pl.ds(s * RPS, RPS)],
                    out_hbm.at[c, pl.ds(s * RPS, RPS)])


@functools.lru_cache(maxsize=1)
def _make_edge():
    return functools.partial(
        pl.kernel,
        out_type=jax.ShapeDtypeStruct((NCORES, N, HT), jnp.float32),
        mesh=plsc.VectorSubcoreMesh(
            core_axis_name="c", subcore_axis_name="s",
            num_cores=NCORES, num_subcores=NSUB),
        scratch_types=[
            pltpu.VMEM((N,), jnp.float32),
            pltpu.VMEM((N,), jnp.float32),
            pltpu.VMEM((NCH + 2, K), jnp.int32),
            pltpu.VMEM((NCH, K), jnp.int32),
            pltpu.VMEM((K, H), jnp.bfloat16),
            pltpu.VMEM((K, H), jnp.bfloat16),
            pltpu.VMEM((K, H), jnp.bfloat16),
            pltpu.VMEM((K, HT), jnp.float32),
            pltpu.VMEM((K, HT), jnp.float32),
            pltpu.VMEM((K, HT), jnp.float32),
            pltpu.VMEM_SHARED((N, HT), jnp.float32),
            pltpu.SemaphoreType.DMA,
            pltpu.SemaphoreType.DMA,
            pltpu.SemaphoreType.DMA,
            pltpu.SemaphoreType.DMA,
            pltpu.SemaphoreType.DMA,
            pltpu.SemaphoreType.DMA,
        ],
        compiler_params=pltpu.CompilerParams(
            use_tc_tiling_on_sc=False, needs_layout_passes=False),
    )(_edge_body)


def kernel(x, edge_index, edge_weight, batch,
           W1, a_s1, a_d1, b1, W2, a_s2, a_d2, b2, Wl, bl, Wc, bc):
    _edge = _make_edge()
    src = jnp.pad(edge_index[0].reshape(NW, NCH, K), ((0, 0), (0, 2), (0, 0)))
    dst = edge_index[1].reshape(NW, NCH, K)
    zeros = jnp.zeros((RPS, HT), jnp.float32)
    gperm = [(m // 2 if m % 2 == 0 else 16 + m // 2) if m < 32 else
             (32 + (m - 32) // 2 if m % 2 == 0 else 48 + (m - 32) // 2)
             for m in range(H)]
    W1p = W1[:, jnp.array(gperm, jnp.int32)]
    W2p = W2[:, jnp.array(gperm, jnp.int32)]

    hb1, as1, ad1 = _dense1(x, W1, W1p, a_s1.reshape(H, 1),
                            a_d1.reshape(H, 1))
    acc1 = _edge(hb1, as1.reshape(N), ad1.reshape(N), src, dst, zeros)
    hb2, as2, ad2 = _dense2(acc1, b1.reshape(1, H), W2, W2p,
                            a_s2.reshape(H, 1), a_d2.reshape(H, 1))
    acc2 = _edge(hb2, as2.reshape(N), ad2.reshape(N), src, dst, zeros)
    log_probs = _final(acc2, b2.reshape(1, H), batch.reshape(1, N),
                       Wl, bl.reshape(1, H // 2), Wc, bc.reshape(1, NCLS))
    return (log_probs, 0)


# speedup vs baseline: 1.2876x; 1.2876x over previous
import functools

import jax
import jax.numpy as jnp
from jax import lax
from jax.experimental import pallas as pl
from jax.experimental.pallas import tpu as pltpu
from jax.experimental.pallas import tpu_sc as plsc

N = 10000
E = 320000
D = 128
H = 64
NG = 16
NCLS = 10

NCORES = 2
NSUB = 16
NW = NCORES * NSUB
EPW = E // NW
K = 80
NCH = EPW // K
HT = 80
RPS = N // NSUB


def _dense1_body(x_ref, w_ref, wp_ref, as_ref, ad_ref,
                 hb_ref, aso_ref, ado_ref):
    h = jnp.dot(x_ref[...], w_ref[...], preferred_element_type=jnp.float32)
    hp = jnp.dot(x_ref[...], wp_ref[...], preferred_element_type=jnp.float32)
    hb_ref[...] = hp.astype(jnp.bfloat16)
    aso_ref[...] = jnp.dot(h, as_ref[...], preferred_element_type=jnp.float32)
    ado_ref[...] = jnp.dot(h, ad_ref[...], preferred_element_type=jnp.float32)


_dense1 = pl.pallas_call(
    _dense1_body,
    out_shape=[
        jax.ShapeDtypeStruct((N, H), jnp.bfloat16),
        jax.ShapeDtypeStruct((N, 1), jnp.float32),
        jax.ShapeDtypeStruct((N, 1), jnp.float32),
    ],
)


def _dense2_body(acc_ref, b_ref, w_ref, wp_ref, as_ref, ad_ref,
                 hb_ref, aso_ref, ado_ref):
    u = acc_ref[0] + acc_ref[1]
    o = u[:, :H] / (u[:, H:H + 1] + 1e-16) + b_ref[...]
    g = jnp.maximum(o, 0.0)
    h = jnp.dot(g, w_ref[...], preferred_element_type=jnp.float32)
    hp = jnp.dot(g, wp_ref[...], preferred_element_type=jnp.float32)
    hb_ref[...] = hp.astype(jnp.bfloat16)
    aso_ref[...] = jnp.dot(h, as_ref[...], preferred_element_type=jnp.float32)
    ado_ref[...] = jnp.dot(h, ad_ref[...], preferred_element_type=jnp.float32)


_dense2 = pl.pallas_call(
    _dense2_body,
    out_shape=[
        jax.ShapeDtypeStruct((N, H), jnp.bfloat16),
        jax.ShapeDtypeStruct((N, 1), jnp.float32),
        jax.ShapeDtypeStruct((N, 1), jnp.float32),
    ],
)


def _final_body(acc_ref, b_ref, batch_ref, wl_ref, bl_ref, wc_ref, bc_ref,
                out_ref):
    u = acc_ref[0] + acc_ref[1]
    o = u[:, :H] / (u[:, H:H + 1] + 1e-16) + b_ref[...]
    gid = lax.broadcasted_iota(jnp.int32, (NG, N), 0)
    m = (gid == batch_ref[...]).astype(jnp.float32)
    sums = jnp.dot(m, o, preferred_element_type=jnp.float32)
    cnt = jnp.sum(m, axis=1, keepdims=True)
    pooled = sums / jnp.maximum(cnt, 1.0)
    z = jnp.maximum(
        jnp.dot(pooled, wl_ref[...], preferred_element_type=jnp.float32)
        + bl_ref[...], 0.0)
    logits = jnp.dot(z, wc_ref[...],
                     preferred_element_type=jnp.float32) + bc_ref[...]
    mx = jnp.max(logits, axis=1, keepdims=True)
    lse = mx + jnp.log(jnp.sum(jnp.exp(logits - mx), axis=1, keepdims=True))
    out_ref[...] = logits - lse


_final = pl.pallas_call(
    _final_body,
    out_shape=jax.ShapeDtypeStruct((NG, NCLS), jnp.float32),
)


def _lane_bcast(v, lane):
    idx = jnp.full((16, 1), lane, jnp.int32)
    return lax.gather(
        v, idx,
        lax.GatherDimensionNumbers(
            offset_dims=(), collapsed_slice_dims=(0,), start_index_map=(0,)),
        (1,), mode=lax.GatherScatterMode.PROMISE_IN_BOUNDS)


def _edge_body(hb_hbm, asrc_hbm, adst_hbm, src_hbm, dst_hbm, zeros_hbm,
               out_hbm, asrc_v, adst_v, srcs_v, dsts_v, raw0, raw1,
               rows0, rows1, accum, sg0, sg1, ss0, ss1):
    c = lax.axis_index("c")
    s = lax.axis_index("s")
    wid = s * NCORES + c
    d0 = pltpu.async_copy(zeros_hbm, accum.at[pl.ds(s * RPS, RPS)], sg0)
    d1 = pltpu.async_copy(asrc_hbm, asrc_v, sg0)
    d2 = pltpu.async_copy(adst_hbm, adst_v, sg0)
    d3 = pltpu.async_copy(src_hbm.at[wid], srcs_v, sg0)
    d4 = pltpu.async_copy(dst_hbm.at[wid], dsts_v, sg0)
    d0.wait(); d1.wait(); d2.wait(); d3.wait(); d4.wait()
    plsc.subcore_barrier()

    raw = (raw0, raw1)
    rows = (rows0, rows1)
    sg = (sg0, sg1)
    ss = (ss0, ss1)

    def gather(j, b):
        return pltpu.async_copy(hb_hbm.at[srcs_v.at[j]], raw[b], sg[b])

    def wvals(j):
        ws = []
        for i in range(K // 16):
            sv = srcs_v[j, pl.ds(i * 16, 16)]
            dv = dsts_v[j, pl.ds(i * 16, 16)]
            e = plsc.load_gather(asrc_v, [sv]) + plsc.load_gather(adst_v, [dv])
            e = jnp.maximum(e, e * 0.2)
            ws.append(jnp.exp(e))
        return ws

    def scale_scatter(j, b, ws):
        mhi = jnp.full((16,), -65536, jnp.int32)
        for r in range(K):
            wsp = _lane_bcast(ws[r // 16], r % 16)
            for q in range(2):
                v = plsc.bitcast(raw[b][r, pl.ds(q * 32, 32)], jnp.int32)
                lo = plsc.bitcast(lax.shift_left(v, 16), jnp.float32)
                hi = plsc.bitcast(lax.bitwise_and(v, mhi), jnp.float32)
                rows[b][r, pl.ds(q * 32, 16)] = lo * wsp
                rows[b][r, pl.ds(q * 32 + 16, 16)] = hi * wsp
            rows[b][r, pl.ds(H, 16)] = wsp
        return pltpu.async_copy(rows[b], accum.at[dsts_v.at[j]], ss[b],
                                add=True)

    def phase(j, b, first, last):
        if not last:
            gather(j + 1, 1 - b)
        ws = wvals(j)
        pltpu.make_async_copy(hb_hbm.at[srcs_v.at[j]], raw[b], sg[b]).wait()
        if not first:
            pltpu.make_async_copy(rows[b], accum.at[dsts_v.at[j - 2]],
                                  ss[b]).wait()
        return scale_scatter(j, b, ws)

    gather(0, 0)
    phase(0, 0, True, False)
    phase(1, 1, True, False)

    def pair(p, carry):
        j = 2 * p
        phase(j, 0, False, False)
        phase(j + 1, 1, False, False)
        return carry

    lax.fori_loop(1, NCH // 2, pair, 0)
    phase(NCH - 1, 0, False, True)
    pltpu.make_async_copy(rows[1], accum.at[dsts_v.at[NCH - 2]], ss[1]).wait()
    pltpu.make_async_copy(rows[0], accum.at[dsts_v.at[NCH - 1]], ss[0]).wait()

    plsc.subcore_barrier()
    pltpu.sync_copy(accum.at[pl.ds(s * RPS, RPS)],
                    out_hbm.at[c, pl.ds(s * RPS, RPS)])


@functools.lru_cache(maxsize=1)
def _make_edge():
    return functools.partial(
        pl.kernel,
        out_type=jax.ShapeDtypeStruct((NCORES, N, HT), jnp.float32),
        mesh=plsc.VectorSubcoreMesh(
            core_axis_name="c", subcore_axis_name="s",
            num_cores=NCORES, num_subcores=NSUB),
        scratch_types=[
            pltpu.VMEM((N,), jnp.float32),
            pltpu.VMEM((N,), jnp.float32),
            pltpu.VMEM((NCH, K), jnp.int32),
            pltpu.VMEM((NCH, K), jnp.int32),
            pltpu.VMEM((K, H), jnp.bfloat16),
            pltpu.VMEM((K, H), jnp.bfloat16),
            pltpu.VMEM((K, HT), jnp.float32),
            pltpu.VMEM((K, HT), jnp.float32),
            pltpu.VMEM_SHARED((N, HT), jnp.float32),
            pltpu.SemaphoreType.DMA,
            pltpu.SemaphoreType.DMA,
            pltpu.SemaphoreType.DMA,
            pltpu.SemaphoreType.DMA,
        ],
        compiler_params=pltpu.CompilerParams(
            use_tc_tiling_on_sc=False, needs_layout_passes=False),
    )(_edge_body)


def kernel(x, edge_index, edge_weight, batch,
           W1, a_s1, a_d1, b1, W2, a_s2, a_d2, b2, Wl, bl, Wc, bc):
    _edge = _make_edge()
    src = edge_index[0].reshape(NW, NCH, K)
    dst = edge_index[1].reshape(NW, NCH, K)
    zeros = jnp.zeros((RPS, HT), jnp.float32)
    gperm = [(m // 2 if m % 2 == 0 else 16 + m // 2) if m < 32 else
             (32 + (m - 32) // 2 if m % 2 == 0 else 48 + (m - 32) // 2)
             for m in range(H)]
    W1p = W1[:, jnp.array(gperm, jnp.int32)]
    W2p = W2[:, jnp.array(gperm, jnp.int32)]

    hb1, as1, ad1 = _dense1(x, W1, W1p, a_s1.reshape(H, 1),
                            a_d1.reshape(H, 1))
    acc1 = _edge(hb1, as1.reshape(N), ad1.reshape(N), src, dst, zeros)
    hb2, as2, ad2 = _dense2(acc1, b1.reshape(1, H), W2, W2p,
                            a_s2.reshape(H, 1), a_d2.reshape(H, 1))
    acc2 = _edge(hb2, as2.reshape(N), ad2.reshape(N), src, dst, zeros)
    log_probs = _final(acc2, b2.reshape(1, H), batch.reshape(1, N),
                       Wl, bl.reshape(1, H // 2), Wc, bc.reshape(1, NCLS))
    return (log_probs, 0)
